# baseline (speedup 1.0000x reference)
import jax
import jax.numpy as jnp
from jax import lax
from jax.experimental import pallas as pl
from jax.experimental.pallas import tpu as pltpu

_N = 25088
_D = 512
_C = 32
_B = 3136
_K = 8
_INV_TEMP = 1.0 / 0.07
_NEG = -1e30


def _fused(s1_ref, lab_ref, s2_ref, sim_ref, out_ref,
           cache_ref, sums_ref, cnt_ref, cent_ref, bias_ref,
           simv_ref, sem):
    i = pl.program_id(0)

    @pl.when(i < _K)
    def _phase_a():
        x = s1_ref[...]
        lab = lab_ref[pl.ds(i, 1), :]
        oh_t = (lax.broadcasted_iota(jnp.int32, (_C, _B), 0)
                == lab).astype(jnp.float32)
        psum = lax.dot_general(oh_t, x, (((1,), (0,)), ((), ())),
                               preferred_element_type=jnp.float32)
        pcnt = jnp.sum(oh_t, axis=1, keepdims=True)

        cache_ref[pl.ds(i * _B, _B), :] = x.astype(jnp.bfloat16)

        @pl.when(i == 0)
        def _init():
            sums_ref[...] = psum
            cnt_ref[...] = pcnt

        @pl.when(i != 0)
        def _acc():
            sums_ref[...] += psum
            cnt_ref[...] += pcnt

    @pl.when(i >= _K)
    def _phase_b():
        j = i - _K
        cp = pltpu.make_async_copy(sim_ref.at[j], simv_ref, sem)
        cp.start()

        @pl.when(i == _K)
        def _finalize():
            cnt = cnt_ref[...]
            recip = 1.0 / jnp.maximum(cnt, 1.0)
            cent_ref[...] = (sums_ref[...] * recip).astype(jnp.bfloat16)
            bias_ref[...] = jnp.where(cnt > 0.0, 0.0, _NEG)

        cent = cent_ref[...]
        bias = bias_ref[...]
        lab = lab_ref[pl.ds(j, 1), :]
        oh_t = lax.broadcasted_iota(jnp.int32, (_C, _B), 0) == lab

        def loss_of(x):
            lg = lax.dot_general(cent, x, (((1,), (1,)), ((), ())),
                                 preferred_element_type=jnp.float32)
            lg = lg * _INV_TEMP + bias
            m = jnp.max(lg, axis=0, keepdims=True)
            lse = jnp.log(jnp.sum(jnp.exp(lg - m), axis=0)) + m[0]
            picked = jnp.sum(jnp.where(oh_t, lg, 0.0), axis=0)
            return lse - picked

        x1 = cache_ref[pl.ds(j * _B, _B), :]
        x2 = s2_ref[...].astype(jnp.bfloat16)
        loss = loss_of(x1) + loss_of(x2)
        cp.wait()
        ones_row = jnp.full((1, 64), 1.0 / 64.0, dtype=jnp.float32)
        w = lax.dot_general(ones_row, simv_ref[...],
                            (((1,), (1,)), ((), ())),
                            preferred_element_type=jnp.float32)[0]
        part = jnp.sum(w * loss) * (0.25 / _N)

        @pl.when(i == _K)
        def _out_init():
            out_ref[0, 0] = part

        @pl.when(i != _K)
        def _out_acc():
            out_ref[0, 0] += part


def kernel(S1, S2, segmentation_map, similarity_matrix):
    out = pl.pallas_call(
        _fused,
        grid=(2 * _K,),
        in_specs=[
            pl.BlockSpec((_B, _D), lambda i: (jnp.minimum(i, _K - 1), 0)),
            pl.BlockSpec(memory_space=pltpu.VMEM),
            pl.BlockSpec((_B, _D), lambda i: (jnp.maximum(i - _K, 0), 0)),
            pl.BlockSpec(memory_space=pltpu.MemorySpace.HBM),
        ],
        out_specs=pl.BlockSpec(memory_space=pltpu.SMEM),
        out_shape=jax.ShapeDtypeStruct((1, 1), jnp.float32),
        scratch_shapes=[
            pltpu.VMEM((_N, _D), jnp.bfloat16),
            pltpu.VMEM((_C, _D), jnp.float32),
            pltpu.VMEM((_C, 1), jnp.float32),
            pltpu.VMEM((_C, _D), jnp.bfloat16),
            pltpu.VMEM((_C, 1), jnp.float32),
            pltpu.VMEM((_B, 64), jnp.float32),
            pltpu.SemaphoreType.DMA,
        ],
        compiler_params=pltpu.CompilerParams(
            dimension_semantics=("arbitrary",)),
    )(S1, segmentation_map, S2, similarity_matrix)

    return out[0, 0]
